# baseline (speedup 1.0000x reference)
import functools

import jax
import jax.numpy as jnp
from jax import lax
from jax.experimental import pallas as pl
from jax.experimental.pallas import tpu as pltpu
from jax.experimental.pallas import tpu_sc as plsc

NC = 2
NS = 16
LANES = 16

B = 80


def _mm_body(h_ref, w_ref, z_ref):
    z_ref[...] = lax.dot_general(
        h_ref[...], w_ref[...], (((1,), (1,)), ((), ())),
        preferred_element_type=jnp.float32)


def _norm_body(s_ref, den_ref, out_ref):
    s = s_ref[0] + s_ref[1]
    d = den_ref[...]
    dsum = d[:, 0] + d[:, 1]
    safe = jnp.where(dsum > 0.0, dsum, 1.0)
    out_ref[...] = s / safe[:, None]


def _sc_body(n_nodes, n_pad, n_edges, z_hbm, ei_hbm, beta_hbm,
             zs_hbm, zd_hbm, s_out, den_out, s_sh, den_sh,
             si0, di0, si1, di1, zs0, zd0, zs1, zd1, exv0, exv1, betav,
             semz0, semz1, semi0, semi1):
    cid = lax.axis_index("c")
    sid = lax.axis_index("s")
    wid = cid * NS + sid

    rpt = (n_nodes // NS) // 8 * 8
    rem = n_nodes - NS * rpt
    row0 = pl.multiple_of(sid * rpt, 8)

    pltpu.sync_copy(zs_hbm.at[pl.ds(row0, rpt)],
                    s_sh.at[pl.ds(row0, rpt)])

    @pl.when(sid == NS - 1)
    def _():
        pltpu.sync_copy(zs_hbm.at[pl.ds(NS * rpt, rem)],
                        s_sh.at[pl.ds(NS * rpt, rem)])

    @pl.when(sid == 0)
    def _():
        pltpu.sync_copy(zd_hbm, den_sh)

    pltpu.sync_copy(beta_hbm, betav)
    bvec = betav[...]
    plsc.subcore_barrier()

    epw = n_edges // (NC * NS)
    nch = epw // B
    base_w = wid * epw
    lanes = lax.iota(jnp.int32, LANES)

    def idx_load(c, si, di, semi):
        base = pl.multiple_of(base_w + c * B, 8)
        base2 = pl.multiple_of(n_edges + base_w + c * B, 8)
        pltpu.async_copy(ei_hbm.at[pl.ds(base, B)], si, semi)
        pltpu.async_copy(ei_hbm.at[pl.ds(base2, B)], di, semi)

    def idx_wait(si, di, semi):
        pltpu.make_async_copy(ei_hbm.at[pl.ds(0, B)], si, semi).wait()
        pltpu.make_async_copy(ei_hbm.at[pl.ds(0, B)], di, semi).wait()

    def gather(si, di, zs, zd, semz):
        pass

    def gather_wait(zs, zd, semz):
        pass

    def compute_chunk(zs, zd, exv, didx):
        def group_body(g, carry2):
            e0 = pl.multiple_of(g * LANES, LANES)
            rows = e0 + lanes

            def k_body(k, acc):
                cols = jnp.zeros((LANES,), jnp.int32) + k
                a = plsc.load_gather(zs, [rows, cols])
                d_ = plsc.load_gather(zd, [rows, cols])
                t = a - d_
                return acc + t * t

            ss = lax.fori_loop(0, zs.shape[1], k_body,
                               jnp.zeros((LANES,), jnp.float32),
                               unroll=8) + 1e-12
            ib = lax.bitcast_convert_type(ss, jnp.int32)
            y = lax.bitcast_convert_type(
                jnp.int32(0x5F3759DF) - (ib >> 1), jnp.float32)
            for _ in range(3):
                y = y * (1.5 - 0.5 * ss * y * y)
            ex = jnp.exp(-bvec * (ss * y))
            exv[pl.ds(e0, LANES)] = ex

            def w_body(k, carry3):
                cols = jnp.zeros((LANES,), jnp.int32) + k
                v = plsc.load_gather(zs, [rows, cols]) * ex
                plsc.store_scatter(zs, [rows, cols], v)
                return carry3

            lax.fori_loop(0, zs.shape[1], w_body, 0, unroll=8)
            return carry2

        lax.fori_loop(0, B // LANES, group_body, 0)

        pltpu.sync_copy(zs, s_sh.at[didx], add=True)
        pltpu.sync_copy(exv, den_sh.at[didx], add=True)

    idx_load(0, si0, di0, semi0)
    idx_wait(si0, di0, semi0)
    gather(si0, di0, zs0, zd0, semz0)
    idx_load(1, si1, di1, semi1)

    def pipe_body(c, carry):
        @pl.when(lax.rem(c, 2) == 0)
        def _():
            gather_wait(zs0, zd0, semz0)

            @pl.when(c + 1 < nch)
            def _():
                idx_wait(si1, di1, semi1)
                gather(si1, di1, zs1, zd1, semz1)
            compute_chunk(zs0, zd0, exv0, di0)

            @pl.when(c + 2 < nch)
            def _():
                idx_load(c + 2, si0, di0, semi0)

        @pl.when(lax.rem(c, 2) == 1)
        def _():
            gather_wait(zs1, zd1, semz1)

            @pl.when(c + 1 < nch)
            def _():
                idx_wait(si0, di0, semi0)
                gather(si0, di0, zs0, zd0, semz0)
            compute_chunk(zs1, zd1, exv1, di1)

            @pl.when(c + 2 < nch)
            def _():
                idx_load(c + 2, si1, di1, semi1)

        return carry

    lax.fori_loop(0, nch, pipe_body, 0)

    plsc.subcore_barrier()

    pltpu.sync_copy(s_sh.at[pl.ds(row0, rpt)],
                    s_out.at[cid, pl.ds(row0, rpt)])

    @pl.when(sid == NS - 1)
    def _():
        pltpu.sync_copy(s_sh.at[pl.ds(NS * rpt, rem)],
                        s_out.at[cid, pl.ds(NS * rpt, rem)])

    @pl.when(sid == 0)
    def _():
        pltpu.sync_copy(den_sh,
                        den_out.at[pl.ds(pl.multiple_of(cid * n_pad, 8),
                                         n_pad)])


def kernel(h, edge_index, W_fc, beta):
    n, in_dim = h.shape
    out_dim = W_fc.shape[0]
    e = edge_index.shape[1]

    row_blk = 1000
    z = pl.pallas_call(
        _mm_body,
        grid=(n // row_blk,),
        in_specs=[
            pl.BlockSpec((row_blk, in_dim), lambda i: (i, 0)),
            pl.BlockSpec((out_dim, in_dim), lambda i: (0, 0)),
        ],
        out_specs=pl.BlockSpec((row_blk, out_dim), lambda i: (i, 0)),
        out_shape=jax.ShapeDtypeStruct((n, out_dim), jnp.float32),
    )(h, W_fc)

    n_pad = (n + 127) // 128 * 128
    zeros2d = jnp.zeros((n, out_dim), jnp.float32)
    zeros1d = jnp.zeros((n_pad,), jnp.float32)
    beta16 = jnp.broadcast_to(beta, (LANES,))

    mesh = plsc.VectorSubcoreMesh(core_axis_name="c", subcore_axis_name="s")
    sc = pl.kernel(
        functools.partial(_sc_body, n, n_pad, e),
        out_type=(
            jax.ShapeDtypeStruct((NC, n, out_dim), jnp.float32),
            jax.ShapeDtypeStruct((NC * n_pad,), jnp.float32),
        ),
        mesh=mesh,
        compiler_params=pltpu.CompilerParams(needs_layout_passes=False),
        scratch_types=[
            pltpu.VMEM_SHARED((n, out_dim), jnp.float32),
            pltpu.VMEM_SHARED((n_pad,), jnp.float32),
            pltpu.VMEM((B,), jnp.int32),
            pltpu.VMEM((B,), jnp.int32),
            pltpu.VMEM((B,), jnp.int32),
            pltpu.VMEM((B,), jnp.int32),
            pltpu.VMEM((B, out_dim), jnp.float32),
            pltpu.VMEM((B, out_dim), jnp.float32),
            pltpu.VMEM((B, out_dim), jnp.float32),
            pltpu.VMEM((B, out_dim), jnp.float32),
            pltpu.VMEM((B,), jnp.float32),
            pltpu.VMEM((B,), jnp.float32),
            pltpu.VMEM((LANES,), jnp.float32),
            pltpu.SemaphoreType.DMA,
            pltpu.SemaphoreType.DMA,
            pltpu.SemaphoreType.DMA,
            pltpu.SemaphoreType.DMA,
        ],
    )
    s_part, den_part = sc(z, edge_index.reshape(-1), beta16, zeros2d, zeros1d)
    den_part = den_part.reshape(NC, n_pad)[:, :n].T

    out = pl.pallas_call(
        _norm_body,
        grid=(n // row_blk,),
        in_specs=[
            pl.BlockSpec((NC, row_blk, out_dim), lambda i: (0, i, 0)),
            pl.BlockSpec((row_blk, NC), lambda i: (i, 0)),
        ],
        out_specs=pl.BlockSpec((row_blk, out_dim), lambda i: (i, 0)),
        out_shape=jax.ShapeDtypeStruct((n, out_dim), jnp.float32),
    )(s_part, den_part)
    return out


# speedup vs baseline: 7.7044x; 7.7044x over previous
import functools

import jax
import jax.numpy as jnp
from jax import lax
from jax.experimental import pallas as pl
from jax.experimental.pallas import tpu as pltpu
from jax.experimental.pallas import tpu_sc as plsc

NC = 2
NS = 16
LANES = 16

B = 80


def _mm_body(h_ref, w_ref, z_ref):
    z_ref[...] = lax.dot_general(
        h_ref[...], w_ref[...], (((1,), (1,)), ((), ())),
        preferred_element_type=jnp.float32)


def _norm_body(s_ref, den_ref, out_ref):
    s = s_ref[0] + s_ref[1]
    d = den_ref[...]
    dsum = d[:, 0] + d[:, 1]
    safe = jnp.where(dsum > 0.0, dsum, 1.0)
    out_ref[...] = s / safe[:, None]


def _sc_body(n_nodes, n_pad, n_edges, z_hbm, ei_hbm, beta_hbm,
             zs_hbm, zd_hbm, s_out, den_out, s_sh, den_sh,
             si0, di0, si1, di1, zs0, zd0, zs1, zd1, exv0, exv1, betav,
             semz0, semz1, semi0, semi1):
    cid = lax.axis_index("c")
    sid = lax.axis_index("s")
    wid = cid * NS + sid

    rpt = (n_nodes // NS) // 8 * 8
    rem = n_nodes - NS * rpt
    row0 = pl.multiple_of(sid * rpt, 8)

    pltpu.sync_copy(zs_hbm.at[pl.ds(row0, rpt)],
                    s_sh.at[pl.ds(row0, rpt)])

    @pl.when(sid == NS - 1)
    def _():
        pltpu.sync_copy(zs_hbm.at[pl.ds(NS * rpt, rem)],
                        s_sh.at[pl.ds(NS * rpt, rem)])

    @pl.when(sid == 0)
    def _():
        pltpu.sync_copy(zd_hbm, den_sh)

    pltpu.sync_copy(beta_hbm, betav)
    bvec = betav[...]
    plsc.subcore_barrier()

    epw = n_edges // (NC * NS)
    nch = epw // B
    base_w = wid * epw
    lanes = lax.iota(jnp.int32, LANES)

    def idx_load(c, si, di, semi):
        base = pl.multiple_of(base_w + c * B, 8)
        base2 = pl.multiple_of(n_edges + base_w + c * B, 8)
        pltpu.async_copy(ei_hbm.at[pl.ds(base, B)], si, semi)
        pltpu.async_copy(ei_hbm.at[pl.ds(base2, B)], di, semi)

    def idx_wait(si, di, semi):
        pltpu.make_async_copy(ei_hbm.at[pl.ds(0, B)], si, semi).wait()
        pltpu.make_async_copy(ei_hbm.at[pl.ds(0, B)], di, semi).wait()

    def gather(si, di, zs, zd, semz):
        pltpu.async_copy(z_hbm.at[si], zs, semz)
        pltpu.async_copy(z_hbm.at[di], zd, semz)

    def gather_wait(zs, zd, semz):
        pltpu.make_async_copy(z_hbm.at[pl.ds(0, B)], zs, semz).wait()
        pltpu.make_async_copy(z_hbm.at[pl.ds(0, B)], zd, semz).wait()

    def compute_chunk(zs, zd, exv, didx):
        def group_body(g, carry2):
            e0 = pl.multiple_of(g * LANES, LANES)
            sums = jnp.zeros((LANES,), jnp.float32)
            for j in range(LANES):
                e = e0 + j
                acc = jnp.zeros((LANES,), jnp.float32)
                for k in range(zs.shape[1] // LANES):
                    a = zs[e, pl.ds(k * LANES, LANES)]
                    d_ = zd[e, pl.ds(k * LANES, LANES)]
                    t = a - d_
                    acc = acc + t * t
                sums = jnp.where(lanes == j, jnp.sum(acc), sums)
            ss = sums + 1e-12
            ib = lax.bitcast_convert_type(ss, jnp.int32)
            y = lax.bitcast_convert_type(
                jnp.int32(0x5F3759DF) - (ib >> 1), jnp.float32)
            for _ in range(3):
                y = y * (1.5 - 0.5 * ss * y * y)
            ex = jnp.exp(-bvec * (ss * y))
            exv[pl.ds(e0, LANES)] = ex
            for j in range(LANES):
                e = e0 + j
                w = ex[j]
                for k in range(zs.shape[1] // LANES):
                    zs[e, pl.ds(k * LANES, LANES)] = (
                        zs[e, pl.ds(k * LANES, LANES)] * w)
            return carry2

        lax.fori_loop(0, B // LANES, group_body, 0)

        pltpu.sync_copy(zs, s_sh.at[didx], add=True)
        pltpu.sync_copy(exv, den_sh.at[didx], add=True)

    idx_load(0, si0, di0, semi0)
    idx_wait(si0, di0, semi0)
    gather(si0, di0, zs0, zd0, semz0)
    idx_load(1, si1, di1, semi1)

    def pipe_body(c, carry):
        @pl.when(lax.rem(c, 2) == 0)
        def _():
            gather_wait(zs0, zd0, semz0)

            @pl.when(c + 1 < nch)
            def _():
                idx_wait(si1, di1, semi1)
                gather(si1, di1, zs1, zd1, semz1)
            compute_chunk(zs0, zd0, exv0, di0)

            @pl.when(c + 2 < nch)
            def _():
                idx_load(c + 2, si0, di0, semi0)

        @pl.when(lax.rem(c, 2) == 1)
        def _():
            gather_wait(zs1, zd1, semz1)

            @pl.when(c + 1 < nch)
            def _():
                idx_wait(si0, di0, semi0)
                gather(si0, di0, zs0, zd0, semz0)
            compute_chunk(zs1, zd1, exv1, di1)

            @pl.when(c + 2 < nch)
            def _():
                idx_load(c + 2, si1, di1, semi1)

        return carry

    lax.fori_loop(0, nch, pipe_body, 0)

    plsc.subcore_barrier()

    pltpu.sync_copy(s_sh.at[pl.ds(row0, rpt)],
                    s_out.at[cid, pl.ds(row0, rpt)])

    @pl.when(sid == NS - 1)
    def _():
        pltpu.sync_copy(s_sh.at[pl.ds(NS * rpt, rem)],
                        s_out.at[cid, pl.ds(NS * rpt, rem)])

    @pl.when(sid == 0)
    def _():
        pltpu.sync_copy(den_sh,
                        den_out.at[pl.ds(pl.multiple_of(cid * n_pad, 8),
                                         n_pad)])


def kernel(h, edge_index, W_fc, beta):
    n, in_dim = h.shape
    out_dim = W_fc.shape[0]
    e = edge_index.shape[1]

    row_blk = 1000
    z = pl.pallas_call(
        _mm_body,
        grid=(n // row_blk,),
        in_specs=[
            pl.BlockSpec((row_blk, in_dim), lambda i: (i, 0)),
            pl.BlockSpec((out_dim, in_dim), lambda i: (0, 0)),
        ],
        out_specs=pl.BlockSpec((row_blk, out_dim), lambda i: (i, 0)),
        out_shape=jax.ShapeDtypeStruct((n, out_dim), jnp.float32),
    )(h, W_fc)

    n_pad = (n + 127) // 128 * 128
    zeros2d = jnp.zeros((n, out_dim), jnp.float32)
    zeros1d = jnp.zeros((n_pad,), jnp.float32)
    beta16 = jnp.broadcast_to(beta, (LANES,))

    mesh = plsc.VectorSubcoreMesh(core_axis_name="c", subcore_axis_name="s")
    sc = pl.kernel(
        functools.partial(_sc_body, n, n_pad, e),
        out_type=(
            jax.ShapeDtypeStruct((NC, n, out_dim), jnp.float32),
            jax.ShapeDtypeStruct((NC * n_pad,), jnp.float32),
        ),
        mesh=mesh,
        compiler_params=pltpu.CompilerParams(needs_layout_passes=False),
        scratch_types=[
            pltpu.VMEM_SHARED((n, out_dim), jnp.float32),
            pltpu.VMEM_SHARED((n_pad,), jnp.float32),
            pltpu.VMEM((B,), jnp.int32),
            pltpu.VMEM((B,), jnp.int32),
            pltpu.VMEM((B,), jnp.int32),
            pltpu.VMEM((B,), jnp.int32),
            pltpu.VMEM((B, out_dim), jnp.float32),
            pltpu.VMEM((B, out_dim), jnp.float32),
            pltpu.VMEM((B, out_dim), jnp.float32),
            pltpu.VMEM((B, out_dim), jnp.float32),
            pltpu.VMEM((B,), jnp.float32),
            pltpu.VMEM((B,), jnp.float32),
            pltpu.VMEM((LANES,), jnp.float32),
            pltpu.SemaphoreType.DMA,
            pltpu.SemaphoreType.DMA,
            pltpu.SemaphoreType.DMA,
            pltpu.SemaphoreType.DMA,
        ],
    )
    s_part, den_part = sc(z, edge_index.reshape(-1), beta16, zeros2d, zeros1d)
    den_part = den_part.reshape(NC, n_pad)[:, :n].T

    out = pl.pallas_call(
        _norm_body,
        grid=(n // row_blk,),
        in_specs=[
            pl.BlockSpec((NC, row_blk, out_dim), lambda i: (0, i, 0)),
            pl.BlockSpec((row_blk, NC), lambda i: (i, 0)),
        ],
        out_specs=pl.BlockSpec((row_blk, out_dim), lambda i: (i, 0)),
        out_shape=jax.ShapeDtypeStruct((n, out_dim), jnp.float32),
    )(s_part, den_part)
    return out
